# baseline (speedup 1.0000x reference)
import functools

import jax
import jax.numpy as jnp
from jax import lax
from jax.experimental import pallas as pl
from jax.experimental.pallas import tpu as pltpu
from jax.experimental.pallas import tpu_sc as plsc

N = 100000
E = 1600000
R = 5
HID = 32
RN = R * N
ZROWS = 6 * N
NC = 2
NS = 16
NP = NC * NS
C = 128
EPT = 102400
E_PAD = EPT * NS
NCHUNK = EPT // C
HEPT = EPT // 2
HALF = N // 2
HALFP = HALF + 16
TRASH = HALF
RPT = 3120
REXTRA = HALF - NS * RPT
CNTP = 512000
CSTRIDE = CNTP // NS
NB = 2 * R
RECTOT = E_PAD + NB * NP * C
SEGTAB = NB * NP * 8
STAGE = 2 * C + 16

_mesh = plsc.VectorSubcoreMesh(core_axis_name="c", subcore_axis_name="s")
_f32 = jnp.float32
_i32 = jnp.int32
_sc_params = pltpu.CompilerParams(use_tc_tiling_on_sc=False,
                                  needs_layout_passes=False)


def _zero_vmem(ref, n):
    def body(i, _):
        ref[pl.ds(i * 16, 16)] = jnp.zeros((16,), _f32)
        return 0
    lax.fori_loop(0, n // 16, body, 0)


def _zero_vmem2(ref, nrows, ncols):
    z = jnp.zeros((16,), _f32)

    def body(i, _):
        for k in range(ncols // 16):
            ref[i, pl.ds(k * 16, 16)] = z
        return 0
    lax.fori_loop(0, nrows, body, 0)


@functools.partial(
    pl.kernel,
    out_type=(
        jax.ShapeDtypeStruct((2 * CNTP,), _f32),
        jax.ShapeDtypeStruct((E_PAD,), _i32),
        jax.ShapeDtypeStruct((NP * 16,), _i32),
    ),
    mesh=_mesh,
    compiler_params=_sc_params,
    scratch_types=[
        pltpu.VMEM((C,), _i32),
        pltpu.VMEM((C,), _i32),
        pltpu.VMEM((C,), _i32),
        pltpu.VMEM((C,), _i32),
        pltpu.VMEM((C,), _i32),
        pltpu.VMEM((C,), _f32),
        pltpu.VMEM((16,), _i32),
        pltpu.VMEM((CSTRIDE,), _f32),
        pltpu.VMEM_SHARED((CNTP,), _f32),
    ],
)
def _p1_counts(src_h, dst_h, et_h, cnt_out, gidx_out, tly_out,
               sv, dv, tv, gv, cv, ones, tbuf, zbuf, cnt_sh):
    c = lax.axis_index("c")
    s = lax.axis_index("s")
    p = 2 * s + c
    _zero_vmem(zbuf, CSTRIDE)
    pltpu.sync_copy(zbuf, cnt_sh.at[pl.ds(s * CSTRIDE, CSTRIDE)])
    for k in range(C // 16):
        ones[pl.ds(k * 16, 16)] = jnp.ones((16,), _f32)
    plsc.subcore_barrier()

    def chunk(i, tly):
        base = p * HEPT + i * C
        pltpu.sync_copy(src_h.at[pl.ds(base, C)], sv)
        pltpu.sync_copy(dst_h.at[pl.ds(base, C)], dv)
        pltpu.sync_copy(et_h.at[pl.ds(base, C)], tv)
        for k in range(C // 16):
            sl = pl.ds(k * 16, 16)
            t = tv[sl]
            d = dv[sl]
            gv[sl] = t * N + sv[sl]
            cv[sl] = jnp.where(d < N, t * N + d, RN)
            bkt = t * 2 + jnp.where(d >= HALF, 1, 0)
            tly = tuple(
                tly[b] + plsc.all_reduce_population_count(bkt == b)[0]
                for b in range(NB))
        pltpu.sync_copy(gv, gidx_out.at[pl.ds(base, C)])
        pltpu.sync_copy(ones, cnt_sh.at[cv], add=True)
        return tly

    tly = lax.fori_loop(0, NCHUNK // 2, chunk, (jnp.int32(0),) * NB)
    ii = lax.iota(_i32, 16)
    tvec = jnp.zeros((16,), _i32)
    for b in range(NB):
        tvec = jnp.where(ii == b, tly[b], tvec)
    tbuf[...] = tvec
    pltpu.sync_copy(tbuf, tly_out.at[pl.ds(p * 16, 16)])
    plsc.subcore_barrier()
    pltpu.sync_copy(cnt_sh.at[pl.ds(s * CSTRIDE, CSTRIDE)],
                    cnt_out.at[pl.ds(c * CNTP + s * CSTRIDE, CSTRIDE)])


def _p2_body(c_ref, inv_ref):
    cnt = c_ref[0] + c_ref[1]
    inv_ref[...] = jnp.maximum(cnt, 1.0)


_p2_inv = pl.pallas_call(
    _p2_body,
    grid=(CNTP // 128 // 8,),
    in_specs=[pl.BlockSpec((2, 8, 128), lambda i: (0, i, 0))],
    out_specs=pl.BlockSpec((8, 128), lambda i: (i, 0)),
    out_shape=jax.ShapeDtypeStruct((CNTP // 128, 128), _f32),
)


@functools.partial(
    pl.kernel,
    out_type=jax.ShapeDtypeStruct((SEGTAB + 8,), _i32),
    mesh=_mesh,
    compiler_params=_sc_params,
    scratch_types=[
        pltpu.VMEM((NP * 16,), _i32),
        pltpu.VMEM((SEGTAB + 16,), _i32),
    ],
)
def _pb_segtab(tly_h, seg_out, cbuf, stab):
    c = lax.axis_index("c")
    s = lax.axis_index("s")

    @pl.when(jnp.logical_and(c == 0, s == 0))
    def _():
        pltpu.sync_copy(tly_h, cbuf)
        ii = lax.iota(_i32, 16)
        vecs = [cbuf[pl.ds(p * 16, 16)] for p in range(NP)]
        running = jnp.int32(0)
        for b in range(NB):
            for p in range(NP):
                cnt = vecs[p][b]
                nch = lax.div(cnt + (C - 1), jnp.int32(C))
                rv = jnp.where(ii == 0, running,
                               jnp.where(ii == 1, nch, 0))
                stab[pl.ds((b * NP + p) * 8, 16)] = rv
                running = running + nch
        pltpu.sync_copy(stab.at[pl.ds(0, SEGTAB + 8)], seg_out)


@functools.partial(
    pl.kernel,
    out_type=(
        jax.ShapeDtypeStruct((RECTOT,), _i32),
        jax.ShapeDtypeStruct((RECTOT,), _i32),
    ),
    mesh=_mesh,
    compiler_params=_sc_params,
    scratch_types=[
        pltpu.VMEM((C,), _i32),
        pltpu.VMEM((C,), _i32),
        pltpu.VMEM((C,), _i32),
        pltpu.VMEM((16,), _i32),
        [pltpu.VMEM((STAGE,), _i32)] * NB,
        [pltpu.VMEM((STAGE,), _i32)] * NB,
    ],
)
def _pc_partition(gidx_h, dst_h, et_h, seg_h, recg_out, recd_out,
                  gvb, dvb, tvb, srow, stg, std):
    c = lax.axis_index("c")
    s = lax.axis_index("s")
    p = 2 * s + c
    woff = []
    for b in range(NB):
        pltpu.sync_copy(seg_h.at[pl.ds((b * NP + p) * 8, 8)],
                        srow.at[pl.ds(0, 8)])
        woff.append(srow[pl.ds(0, 16)][0])

    def chunk(i, carry):
        wpos = list(carry[:NB])
        wcur = list(carry[NB:])
        base = p * HEPT + i * C
        pltpu.sync_copy(gidx_h.at[pl.ds(base, C)], gvb)
        pltpu.sync_copy(dst_h.at[pl.ds(base, C)], dvb)
        pltpu.sync_copy(et_h.at[pl.ds(base, C)], tvb)
        for k in range(C // 16):
            sl = pl.ds(k * 16, 16)
            g = gvb[sl]
            d = dvb[sl]
            t = tvb[sl]
            h1 = jnp.where(d >= HALF, 1, 0)
            bkt = t * 2 + h1
            dl = d - h1 * HALF
            for b in range(NB):
                m = bkt == b
                plsc.store_compressed(stg[b].at[pl.ds(wpos[b], 16)], g,
                                      mask=m)
                plsc.store_compressed(std[b].at[pl.ds(wpos[b], 16)], dl,
                                      mask=m)
                wpos[b] = wpos[b] + plsc.all_reduce_population_count(m)[0]
        for b in range(NB):
            full = wpos[b] >= C

            @pl.when(full)
            def _(b=b):
                pltpu.sync_copy(stg[b].at[pl.ds(0, C)],
                                recg_out.at[pl.ds(wcur[b] * C, C)])
                pltpu.sync_copy(std[b].at[pl.ds(0, C)],
                                recd_out.at[pl.ds(wcur[b] * C, C)])
                for j in range(C // 16):
                    stg[b][pl.ds(j * 16, 16)] = stg[b][pl.ds(C + j * 16, 16)]
                    std[b][pl.ds(j * 16, 16)] = std[b][pl.ds(C + j * 16, 16)]

            wpos[b] = jnp.where(full, wpos[b] - C, wpos[b])
            wcur[b] = jnp.where(full, wcur[b] + 1, wcur[b])
        return tuple(wpos) + tuple(wcur)

    init = tuple(jnp.int32(0) for _ in range(NB)) + tuple(woff)
    carry = lax.fori_loop(0, HEPT // C, chunk, init)
    wpos = carry[:NB]
    wcur = carry[NB:]
    trg = jnp.zeros((16,), _i32)
    trd = jnp.full((16,), TRASH, _i32)
    for b in range(NB):
        for j in range(C // 16):
            off = wpos[b] + j * 16
            stg[b][pl.ds(off, 16)] = trg
            std[b][pl.ds(off, 16)] = trd

        @pl.when(wpos[b] > 0)
        def _(b=b):
            pltpu.sync_copy(stg[b].at[pl.ds(0, C)],
                            recg_out.at[pl.ds(wcur[b] * C, C)])
            pltpu.sync_copy(std[b].at[pl.ds(0, C)],
                            recd_out.at[pl.ds(wcur[b] * C, C)])


_bf16 = jnp.bfloat16


def _r16(a):
    return a.astype(_bf16).astype(_f32)


def _dot16(a, b):
    return jnp.dot(a.astype(_bf16), b.astype(_bf16),
                   preferred_element_type=_f32)


def _wcomb(basis_ref, cm):
    return (_r16(cm[0]) * _r16(basis_ref[0])
            + _r16(cm[1]) * _r16(basis_ref[1])
            + _r16(cm[2]) * _r16(basis_ref[2]))


def _make_proj(nin, apply_tanh):
    def body(h_ref, basis_ref, comp_ref, b_ref, z_ref):
        cm = comp_ref[0, 0]
        w = _wcomb(basis_ref, cm)
        h = h_ref[...]
        if apply_tanh:
            h = jnp.tanh(h)
        z_ref[...] = _dot16(h, w) + b_ref[0, 0]

    blk = 1000
    nb = N // blk
    return pl.pallas_call(
        body,
        grid=(6, nb),
        in_specs=[
            pl.BlockSpec((blk, nin), lambda i, j: (j, 0)),
            pl.BlockSpec((3, nin, HID), lambda i, j: (0, 0, 0)),
            pl.BlockSpec((1, 1, 3), lambda i, j: (i, 0, 0)),
            pl.BlockSpec((1, 1, HID), lambda i, j: (i, 0, 0)),
        ],
        out_specs=pl.BlockSpec((blk, HID), lambda i, j: (i * nb + j, 0)),
        out_shape=jax.ShapeDtypeStruct((ZROWS, HID), _f32),
    )


_proj_first = _make_proj(4, False)

_BLK = 1000
_NBLK = N // _BLK


def _combine(zself_ref, agg_ref, cnt_ref):
    acc = zself_ref[...]
    for r in range(R):
        acc = acc + agg_ref[r] / cnt_ref[r]
    return acc


def _projf_body(zp_ref, agg_ref, cnt_ref, basis_ref, comp_ref, b_ref,
                z_ref, pre_ref):
    pre = _combine(zp_ref, agg_ref, cnt_ref)
    pre_ref[...] = pre
    cm = comp_ref[0, 0]
    w = _wcomb(basis_ref, cm)
    z_ref[...] = _dot16(jnp.tanh(pre), w) + b_ref[0, 0]


_proj_fused = pl.pallas_call(
    _projf_body,
    grid=(6, _NBLK),
    in_specs=[
        pl.BlockSpec((_BLK, HID), lambda i, j: (RN // _BLK + j, 0)),
        pl.BlockSpec((R, _BLK, HID), lambda i, j: (0, j, 0)),
        pl.BlockSpec((R, _BLK, 1), lambda i, j: (0, j, 0)),
        pl.BlockSpec((3, HID, HID), lambda i, j: (0, 0, 0)),
        pl.BlockSpec((1, 1, 3), lambda i, j: (i, 0, 0)),
        pl.BlockSpec((1, 1, HID), lambda i, j: (i, 0, 0)),
    ],
    out_specs=[
        pl.BlockSpec((_BLK, HID), lambda i, j: (i * _NBLK + j, 0)),
        pl.BlockSpec((_BLK, HID), lambda i, j: (j, 0)),
    ],
    out_shape=[
        jax.ShapeDtypeStruct((ZROWS, HID), _f32),
        jax.ShapeDtypeStruct((N, HID), _f32),
    ],
)


def _cmb_body(zp_ref, agg_ref, inv_ref, out_ref):
    out_ref[...] = _combine(zp_ref, agg_ref, inv_ref)


_cmb = pl.pallas_call(
    _cmb_body,
    grid=(_NBLK,),
    in_specs=[
        pl.BlockSpec((_BLK, HID), lambda j: (RN // _BLK + j, 0)),
        pl.BlockSpec((R, _BLK, HID), lambda j: (0, j, 0)),
        pl.BlockSpec((R, _BLK, 1), lambda j: (0, j, 0)),
    ],
    out_specs=pl.BlockSpec((_BLK, HID), lambda j: (j, 0)),
    out_shape=jax.ShapeDtypeStruct((N, HID), _f32),
)


ZB = 208


@functools.partial(
    pl.kernel,
    out_type=jax.ShapeDtypeStruct((RN, HID), _f32),
    mesh=_mesh,
    compiler_params=_sc_params,
    scratch_types=[
        [pltpu.VMEM((C,), _i32)] * 2,
        [pltpu.VMEM((C,), _i32)] * 2,
        [pltpu.VMEM((C, HID), _f32)] * 2,
        pltpu.VMEM((16,), _i32),
        pltpu.VMEM((ZB, HID), _f32),
        pltpu.VMEM_SHARED((HALFP, HID), _f32),
        [pltpu.SemaphoreType.DMA] * 2,
        [pltpu.SemaphoreType.DMA] * 2,
    ],
)
def _agg(z_h, recg_h, recd_h, seg_h, out_h,
         gv, dlv, rows, srow, zbuf, acc, sem_r, sem_g):
    c = lax.axis_index("c")
    s = lax.axis_index("s")
    lo = c * HALF
    _zero_vmem2(zbuf, ZB, HID)

    for r in range(R):
        for j in range(RPT // ZB):
            pltpu.sync_copy(zbuf, acc.at[pl.ds(s * RPT + j * ZB, ZB)])

        @pl.when(s == 0)
        def _():
            pltpu.sync_copy(zbuf.at[pl.ds(0, REXTRA + 16)],
                            acc.at[pl.ds(NS * RPT, REXTRA + 16)])

        plsc.subcore_barrier()

        segs = []
        for j in range(2):
            row = ((r * 2) * NP + c * NP + 2 * s + j) * 8
            pltpu.sync_copy(seg_h.at[pl.ds(row, 8)], srow.at[pl.ds(0, 8)])
            v = srow[pl.ds(0, 16)]
            segs.append((v[0], v[1]))
        (off0, n0), (off1, n1) = segs
        ntot = n0 + n1

        def rbase(i):
            return jnp.where(i < n0, off0 + i, off1 + (i - n0)) * C

        def rec_load(i, b):
            base = rbase(i)
            pltpu.async_copy(recg_h.at[pl.ds(base, C)], gv[b], sem_r[b])
            pltpu.async_copy(recd_h.at[pl.ds(base, C)], dlv[b], sem_r[b])

        def rec_wait(i, b):
            base = rbase(i)
            pltpu.make_async_copy(recg_h.at[pl.ds(base, C)], gv[b],
                                  sem_r[b]).wait()
            pltpu.make_async_copy(recd_h.at[pl.ds(base, C)], dlv[b],
                                  sem_r[b]).wait()

        @pl.when(ntot > 0)
        def _():
            rec_load(0, 0)

        @pl.when(ntot > 1)
        def _():
            rec_load(1, 1)

        @pl.when(ntot > 0)
        def _():
            rec_wait(0, 0)
            pltpu.async_copy(z_h.at[gv[0]], rows[0], sem_g[0])

        def pair(i2, _):
            for q in (0, 1):
                i = i2 * 2 + q
                o = 1 - q

                @pl.when(i < ntot)
                def _(i=i, q=q, o=o):
                    pltpu.make_async_copy(z_h.at[gv[q]], rows[q],
                                          sem_g[q]).wait()

                    @pl.when(i + 1 < ntot)
                    def _():
                        rec_wait(i + 1, o)
                        pltpu.async_copy(z_h.at[gv[o]], rows[o], sem_g[o])

                    pltpu.sync_copy(rows[q], acc.at[dlv[q]], add=True)

                    @pl.when(i + 2 < ntot)
                    def _():
                        rec_load(i + 2, q)

            return 0

        lax.fori_loop(0, lax.div(ntot + 1, jnp.int32(2)), pair, 0)
        plsc.subcore_barrier()
        pltpu.sync_copy(acc.at[pl.ds(s * RPT, RPT)],
                        out_h.at[pl.ds(r * N + lo + s * RPT, RPT)])

        @pl.when(s == 0)
        def _():
            pltpu.sync_copy(acc.at[pl.ds(NS * RPT, REXTRA)],
                            out_h.at[pl.ds(r * N + lo + NS * RPT, REXTRA)])


def _mlp_body(u0, u1, u2, u3, i0, i1, i2, i3, w1_ref, b1_ref, w2_ref, b2_ref,
              out_ref):
    feats = [u0, u1, u2, u3, i0, i1, i2, i3]
    acc = jnp.broadcast_to(b1_ref[...], (1000, 128))
    for k, f in enumerate(feats):
        h = jnp.tanh(f[...])
        acc = acc + _dot16(h, w1_ref[pl.ds(32 * k, 32), :])
    r = jnp.maximum(acc, 0.0)
    o = (jnp.sum(_r16(r) * _r16(w2_ref[...]), axis=1, keepdims=True)
         + b2_ref[0, 0])
    out_ref[...] = o


def _mlp(us, its, w1, b1, w2t, b2):
    nq = N // 4
    specs = [pl.BlockSpec((1000, HID), lambda i: (i, 0))] * 8
    specs += [
        pl.BlockSpec((256, 128), lambda i: (0, 0)),
        pl.BlockSpec((1, 128), lambda i: (0, 0)),
        pl.BlockSpec((1, 128), lambda i: (0, 0)),
        pl.BlockSpec((1, 1), lambda i: (0, 0)),
    ]
    return pl.pallas_call(
        _mlp_body,
        grid=(nq // 1000,),
        in_specs=specs,
        out_specs=pl.BlockSpec((1000, 1), lambda i: (i, 0)),
        out_shape=jax.ShapeDtypeStruct((nq, 1), _f32),
    )(*us, *its, w1, b1, w2t, b2)


def kernel(x, edge_index, edge_type, batch,
           basis0, comp0, root0, bias0, basis1, comp1, root1, bias1,
           basis2, comp2, root2, bias2, basis3, comp3, root3, bias3,
           W1, b1, W2, b2):
    src = edge_index[0]
    dst = edge_index[1]
    npad = E_PAD - E
    src_p = jnp.concatenate([src, jnp.zeros((npad,), _i32)])
    dst_p = jnp.concatenate([dst, jnp.full((npad,), N, _i32)])
    et_p = jnp.concatenate([edge_type, jnp.zeros((npad,), _i32)])

    cnt_part, gidx, tly = _p1_counts(src_p, dst_p, et_p)
    inv = _p2_inv(cnt_part.reshape(2, CNTP // 128, 128)).reshape(CNTP)
    invr = inv[:RN].reshape(R, N, 1)
    segtab = _pb_segtab(tly)
    recg, recd = _pc_partition(gidx, dst_p, et_p, segtab)

    def wprep(basis, comp, root, bias):
        b6 = jnp.concatenate([basis, root[None]], axis=0)
        c6 = jnp.concatenate([
            jnp.concatenate([comp, jnp.zeros((R, 1), _f32)], axis=1),
            jnp.array([[0.0, 0.0, 1.0]], _f32)], axis=0).reshape(6, 1, 3)
        bb = jnp.concatenate([jnp.zeros((R, HID), _f32), bias[None]],
                             axis=0).reshape(6, 1, HID)
        return b6, c6, bb

    layers = [(basis0, comp0, root0, bias0), (basis1, comp1, root1, bias1),
              (basis2, comp2, root2, bias2), (basis3, comp3, root3, bias3)]

    b6, c6, bb = wprep(*layers[0])
    z = _proj_first(x, b6, c6, bb)
    agg = _agg(z, recg, recd, segtab).reshape(R, N, HID)
    pre = []
    for li in (1, 2, 3):
        b6, c6, bb = wprep(*layers[li])
        z_new, p_prev = _proj_fused(z, agg, invr, b6, c6, bb)
        pre.append(p_prev)
        z = z_new
        agg = _agg(z, recg, recd, segtab).reshape(R, N, HID)
    pre.append(_cmb(z, agg, invr))

    nq = N // 4
    us = [p.reshape(nq, 4, HID)[:, 0, :] for p in pre]
    its = [p.reshape(nq, 4, HID)[:, 1, :] for p in pre]
    return _mlp(us, its, W1, b1.reshape(1, 128), W2.reshape(1, 128),
                b2.reshape(1, 1))


# speedup vs baseline: 1.2737x; 1.2737x over previous
import functools

import jax
import jax.numpy as jnp
from jax import lax
from jax.experimental import pallas as pl
from jax.experimental.pallas import tpu as pltpu
from jax.experimental.pallas import tpu_sc as plsc

N = 100000
E = 1600000
R = 5
HID = 32
RN = R * N
ZROWS = 6 * N
NC = 2
NS = 16
NP = NC * NS
C = 128
EPT = 102400
E_PAD = EPT * NS
NCHUNK = EPT // C
HEPT = EPT // 2
HALF = N // 2
HALFP = HALF + 16
TRASH = HALF
RPT = 3120
REXTRA = HALF - NS * RPT
CNTP = 512000
CSTRIDE = CNTP // NS
NB = 2 * R
RECTOT = E_PAD + NB * NP * C
SEGTAB = NB * NP * 8
STAGE = 2 * C + 16

_mesh = plsc.VectorSubcoreMesh(core_axis_name="c", subcore_axis_name="s")
_f32 = jnp.float32
_i32 = jnp.int32
_sc_params = pltpu.CompilerParams(use_tc_tiling_on_sc=False,
                                  needs_layout_passes=False)


def _zero_vmem(ref, n):
    def body(i, _):
        ref[pl.ds(i * 16, 16)] = jnp.zeros((16,), _f32)
        return 0
    lax.fori_loop(0, n // 16, body, 0)


def _zero_vmem2(ref, nrows, ncols):
    z = jnp.zeros((16,), _f32)

    def body(i, _):
        for k in range(ncols // 16):
            ref[i, pl.ds(k * 16, 16)] = z
        return 0
    lax.fori_loop(0, nrows, body, 0)


@functools.partial(
    pl.kernel,
    out_type=(
        jax.ShapeDtypeStruct((2 * CNTP,), _f32),
        jax.ShapeDtypeStruct((E_PAD,), _i32),
        jax.ShapeDtypeStruct((NP * 16,), _i32),
    ),
    mesh=_mesh,
    compiler_params=_sc_params,
    scratch_types=[
        pltpu.VMEM((C,), _i32),
        pltpu.VMEM((C,), _i32),
        pltpu.VMEM((C,), _i32),
        pltpu.VMEM((C,), _i32),
        pltpu.VMEM((C,), _i32),
        pltpu.VMEM((C,), _f32),
        pltpu.VMEM((16,), _i32),
        pltpu.VMEM((CSTRIDE,), _f32),
        pltpu.VMEM_SHARED((CNTP,), _f32),
    ],
)
def _p1_counts(src_h, dst_h, et_h, cnt_out, gidx_out, tly_out,
               sv, dv, tv, gv, cv, ones, tbuf, zbuf, cnt_sh):
    c = lax.axis_index("c")
    s = lax.axis_index("s")
    p = 2 * s + c
    _zero_vmem(zbuf, CSTRIDE)
    pltpu.sync_copy(zbuf, cnt_sh.at[pl.ds(s * CSTRIDE, CSTRIDE)])
    for k in range(C // 16):
        ones[pl.ds(k * 16, 16)] = jnp.ones((16,), _f32)
    plsc.subcore_barrier()

    def chunk(i, tly):
        base = p * HEPT + i * C
        pltpu.sync_copy(src_h.at[pl.ds(base, C)], sv)
        pltpu.sync_copy(dst_h.at[pl.ds(base, C)], dv)
        pltpu.sync_copy(et_h.at[pl.ds(base, C)], tv)
        for k in range(C // 16):
            sl = pl.ds(k * 16, 16)
            t = tv[sl]
            d = dv[sl]
            gv[sl] = t * N + sv[sl]
            cv[sl] = jnp.where(d < N, t * N + d, RN)
            bkt = jnp.where(d < N, t * 2 + jnp.where(d >= HALF, 1, 0), NB)
            tly = tuple(
                tly[b] + plsc.all_reduce_population_count(bkt == b)[0]
                for b in range(NB))
        pltpu.sync_copy(gv, gidx_out.at[pl.ds(base, C)])
        pltpu.sync_copy(ones, cnt_sh.at[cv], add=True)
        return tly

    tly = lax.fori_loop(0, NCHUNK // 2, chunk, (jnp.int32(0),) * NB)
    ii = lax.iota(_i32, 16)
    tvec = jnp.zeros((16,), _i32)
    for b in range(NB):
        tvec = jnp.where(ii == b, tly[b], tvec)
    tbuf[...] = tvec
    pltpu.sync_copy(tbuf, tly_out.at[pl.ds(p * 16, 16)])
    plsc.subcore_barrier()
    pltpu.sync_copy(cnt_sh.at[pl.ds(s * CSTRIDE, CSTRIDE)],
                    cnt_out.at[pl.ds(c * CNTP + s * CSTRIDE, CSTRIDE)])


def _p2_body(c_ref, inv_ref):
    cnt = c_ref[0] + c_ref[1]
    inv_ref[...] = jnp.maximum(cnt, 1.0)


_p2_inv = pl.pallas_call(
    _p2_body,
    grid=(CNTP // 128 // 8,),
    in_specs=[pl.BlockSpec((2, 8, 128), lambda i: (0, i, 0))],
    out_specs=pl.BlockSpec((8, 128), lambda i: (i, 0)),
    out_shape=jax.ShapeDtypeStruct((CNTP // 128, 128), _f32),
)


@functools.partial(
    pl.kernel,
    out_type=jax.ShapeDtypeStruct((SEGTAB + 8,), _i32),
    mesh=_mesh,
    compiler_params=_sc_params,
    scratch_types=[
        pltpu.VMEM((NP * 16,), _i32),
        pltpu.VMEM((SEGTAB + 16,), _i32),
    ],
)
def _pb_segtab(tly_h, seg_out, cbuf, stab):
    c = lax.axis_index("c")
    s = lax.axis_index("s")

    @pl.when(jnp.logical_and(c == 0, s == 0))
    def _():
        pltpu.sync_copy(tly_h, cbuf)
        ii = lax.iota(_i32, 16)
        vecs = [cbuf[pl.ds(p * 16, 16)] for p in range(NP)]
        running = jnp.int32(0)
        for b in range(NB):
            for p in range(NP):
                cnt = vecs[p][b]
                nch = lax.div(cnt + (C - 1), jnp.int32(C))
                rv = jnp.where(ii == 0, running,
                               jnp.where(ii == 1, nch, 0))
                stab[pl.ds((b * NP + p) * 8, 16)] = rv
                running = running + nch
        pltpu.sync_copy(stab.at[pl.ds(0, SEGTAB + 8)], seg_out)


@functools.partial(
    pl.kernel,
    out_type=(
        jax.ShapeDtypeStruct((RECTOT,), _i32),
        jax.ShapeDtypeStruct((RECTOT,), _i32),
    ),
    mesh=_mesh,
    compiler_params=_sc_params,
    scratch_types=[
        pltpu.VMEM((C,), _i32),
        pltpu.VMEM((C,), _i32),
        pltpu.VMEM((C,), _i32),
        pltpu.VMEM((16,), _i32),
        [pltpu.VMEM((STAGE,), _i32)] * NB,
        [pltpu.VMEM((STAGE,), _i32)] * NB,
    ],
)
def _pc_partition(gidx_h, dst_h, et_h, seg_h, recg_out, recd_out,
                  gvb, dvb, tvb, srow, stg, std):
    c = lax.axis_index("c")
    s = lax.axis_index("s")
    p = 2 * s + c
    woff = []
    for b in range(NB):
        pltpu.sync_copy(seg_h.at[pl.ds((b * NP + p) * 8, 8)],
                        srow.at[pl.ds(0, 8)])
        woff.append(srow[pl.ds(0, 16)][0])

    def chunk(i, carry):
        wpos = list(carry[:NB])
        wcur = list(carry[NB:])
        base = p * HEPT + i * C
        pltpu.sync_copy(gidx_h.at[pl.ds(base, C)], gvb)
        pltpu.sync_copy(dst_h.at[pl.ds(base, C)], dvb)
        pltpu.sync_copy(et_h.at[pl.ds(base, C)], tvb)
        for k in range(C // 16):
            sl = pl.ds(k * 16, 16)
            g = gvb[sl]
            d = dvb[sl]
            t = tvb[sl]
            h1 = jnp.where(d >= HALF, 1, 0)
            bkt = jnp.where(d < N, t * 2 + h1, NB)
            dl = jnp.minimum(d - h1 * HALF, TRASH)
            for b in range(NB):
                m = bkt == b
                plsc.store_compressed(stg[b].at[pl.ds(wpos[b], 16)], g,
                                      mask=m)
                plsc.store_compressed(std[b].at[pl.ds(wpos[b], 16)], dl,
                                      mask=m)
                wpos[b] = wpos[b] + plsc.all_reduce_population_count(m)[0]
        for b in range(NB):
            full = wpos[b] >= C

            @pl.when(full)
            def _(b=b):
                pltpu.sync_copy(stg[b].at[pl.ds(0, C)],
                                recg_out.at[pl.ds(wcur[b] * C, C)])
                pltpu.sync_copy(std[b].at[pl.ds(0, C)],
                                recd_out.at[pl.ds(wcur[b] * C, C)])
                for j in range(C // 16):
                    stg[b][pl.ds(j * 16, 16)] = stg[b][pl.ds(C + j * 16, 16)]
                    std[b][pl.ds(j * 16, 16)] = std[b][pl.ds(C + j * 16, 16)]

            wpos[b] = jnp.where(full, wpos[b] - C, wpos[b])
            wcur[b] = jnp.where(full, wcur[b] + 1, wcur[b])
        return tuple(wpos) + tuple(wcur)

    init = tuple(jnp.int32(0) for _ in range(NB)) + tuple(woff)
    carry = lax.fori_loop(0, HEPT // C, chunk, init)
    wpos = carry[:NB]
    wcur = carry[NB:]
    trg = jnp.zeros((16,), _i32)
    trd = jnp.full((16,), TRASH, _i32)
    for b in range(NB):
        for j in range(C // 16):
            off = wpos[b] + j * 16
            stg[b][pl.ds(off, 16)] = trg
            std[b][pl.ds(off, 16)] = trd

        @pl.when(wpos[b] > 0)
        def _(b=b):
            pltpu.sync_copy(stg[b].at[pl.ds(0, C)],
                            recg_out.at[pl.ds(wcur[b] * C, C)])
            pltpu.sync_copy(std[b].at[pl.ds(0, C)],
                            recd_out.at[pl.ds(wcur[b] * C, C)])


_bf16 = jnp.bfloat16


def _r16(a):
    return a.astype(_bf16).astype(_f32)


def _dot16(a, b):
    return jnp.dot(a.astype(_bf16), b.astype(_bf16),
                   preferred_element_type=_f32)


def _wcomb(basis_ref, cm):
    return (_r16(cm[0]) * _r16(basis_ref[0])
            + _r16(cm[1]) * _r16(basis_ref[1])
            + _r16(cm[2]) * _r16(basis_ref[2]))


def _make_proj(nin, apply_tanh):
    def body(h_ref, basis_ref, comp_ref, b_ref, z_ref):
        cm = comp_ref[0, 0]
        w = _wcomb(basis_ref, cm)
        h = h_ref[...]
        if apply_tanh:
            h = jnp.tanh(h)
        z_ref[...] = _dot16(h, w) + b_ref[0, 0]

    blk = 1000
    nb = N // blk
    return pl.pallas_call(
        body,
        grid=(6, nb),
        in_specs=[
            pl.BlockSpec((blk, nin), lambda i, j: (j, 0)),
            pl.BlockSpec((3, nin, HID), lambda i, j: (0, 0, 0)),
            pl.BlockSpec((1, 1, 3), lambda i, j: (i, 0, 0)),
            pl.BlockSpec((1, 1, HID), lambda i, j: (i, 0, 0)),
        ],
        out_specs=pl.BlockSpec((blk, HID), lambda i, j: (i * nb + j, 0)),
        out_shape=jax.ShapeDtypeStruct((ZROWS, HID), _f32),
    )


_proj_first = _make_proj(4, False)

_BLK = 1000
_NBLK = N // _BLK


def _combine(zself_ref, agg_ref, cnt_ref):
    acc = zself_ref[...]
    for r in range(R):
        acc = acc + agg_ref[r] / cnt_ref[r]
    return acc


def _projf_body(zp_ref, agg_ref, cnt_ref, basis_ref, comp_ref, b_ref,
                z_ref, pre_ref):
    pre = _combine(zp_ref, agg_ref, cnt_ref)
    pre_ref[...] = pre
    cm = comp_ref[0, 0]
    w = _wcomb(basis_ref, cm)
    z_ref[...] = _dot16(jnp.tanh(pre), w) + b_ref[0, 0]


_proj_fused = pl.pallas_call(
    _projf_body,
    grid=(6, _NBLK),
    in_specs=[
        pl.BlockSpec((_BLK, HID), lambda i, j: (RN // _BLK + j, 0)),
        pl.BlockSpec((R, _BLK, HID), lambda i, j: (0, j, 0)),
        pl.BlockSpec((R, _BLK, 1), lambda i, j: (0, j, 0)),
        pl.BlockSpec((3, HID, HID), lambda i, j: (0, 0, 0)),
        pl.BlockSpec((1, 1, 3), lambda i, j: (i, 0, 0)),
        pl.BlockSpec((1, 1, HID), lambda i, j: (i, 0, 0)),
    ],
    out_specs=[
        pl.BlockSpec((_BLK, HID), lambda i, j: (i * _NBLK + j, 0)),
        pl.BlockSpec((_BLK, HID), lambda i, j: (j, 0)),
    ],
    out_shape=[
        jax.ShapeDtypeStruct((ZROWS, HID), _f32),
        jax.ShapeDtypeStruct((N, HID), _f32),
    ],
)


def _cmb_body(zp_ref, agg_ref, inv_ref, out_ref):
    out_ref[...] = _combine(zp_ref, agg_ref, inv_ref)


_cmb = pl.pallas_call(
    _cmb_body,
    grid=(_NBLK,),
    in_specs=[
        pl.BlockSpec((_BLK, HID), lambda j: (RN // _BLK + j, 0)),
        pl.BlockSpec((R, _BLK, HID), lambda j: (0, j, 0)),
        pl.BlockSpec((R, _BLK, 1), lambda j: (0, j, 0)),
    ],
    out_specs=pl.BlockSpec((_BLK, HID), lambda j: (j, 0)),
    out_shape=jax.ShapeDtypeStruct((N, HID), _f32),
)


ZB = 208


@functools.partial(
    pl.kernel,
    out_type=jax.ShapeDtypeStruct((RN, HID), _f32),
    mesh=_mesh,
    compiler_params=_sc_params,
    scratch_types=[
        [pltpu.VMEM((C,), _i32)] * 2,
        [pltpu.VMEM((C,), _i32)] * 2,
        [pltpu.VMEM((C, HID), _f32)] * 2,
        pltpu.VMEM((16,), _i32),
        pltpu.VMEM((ZB, HID), _f32),
        pltpu.VMEM_SHARED((HALFP, HID), _f32),
        [pltpu.SemaphoreType.DMA] * 2,
        [pltpu.SemaphoreType.DMA] * 2,
    ],
)
def _agg(z_h, recg_h, recd_h, seg_h, out_h,
         gv, dlv, rows, srow, zbuf, acc, sem_r, sem_g):
    c = lax.axis_index("c")
    s = lax.axis_index("s")
    lo = c * HALF
    _zero_vmem2(zbuf, ZB, HID)

    for r in range(R):
        for j in range(RPT // ZB):
            pltpu.sync_copy(zbuf, acc.at[pl.ds(s * RPT + j * ZB, ZB)])

        @pl.when(s == 0)
        def _():
            pltpu.sync_copy(zbuf.at[pl.ds(0, REXTRA + 16)],
                            acc.at[pl.ds(NS * RPT, REXTRA + 16)])

        plsc.subcore_barrier()

        segs = []
        for j in range(2):
            row = ((r * 2) * NP + c * NP + 2 * s + j) * 8
            pltpu.sync_copy(seg_h.at[pl.ds(row, 8)], srow.at[pl.ds(0, 8)])
            v = srow[pl.ds(0, 16)]
            segs.append((v[0], v[1]))
        (off0, n0), (off1, n1) = segs
        ntot = n0 + n1

        def rbase(i):
            return jnp.where(i < n0, off0 + i, off1 + (i - n0)) * C

        def rec_load(i, b):
            base = rbase(i)
            pltpu.async_copy(recg_h.at[pl.ds(base, C)], gv[b], sem_r[b])
            pltpu.async_copy(recd_h.at[pl.ds(base, C)], dlv[b], sem_r[b])

        def rec_wait(i, b):
            base = rbase(i)
            pltpu.make_async_copy(recg_h.at[pl.ds(base, C)], gv[b],
                                  sem_r[b]).wait()
            pltpu.make_async_copy(recd_h.at[pl.ds(base, C)], dlv[b],
                                  sem_r[b]).wait()

        @pl.when(ntot > 0)
        def _():
            rec_load(0, 0)

        @pl.when(ntot > 1)
        def _():
            rec_load(1, 1)

        @pl.when(ntot > 0)
        def _():
            rec_wait(0, 0)
            pltpu.async_copy(z_h.at[gv[0]], rows[0], sem_g[0])

        def pair(i2, _):
            for q in (0, 1):
                i = i2 * 2 + q
                o = 1 - q

                @pl.when(i < ntot)
                def _(i=i, q=q, o=o):
                    pltpu.make_async_copy(z_h.at[gv[q]], rows[q],
                                          sem_g[q]).wait()

                    @pl.when(i + 1 < ntot)
                    def _():
                        rec_wait(i + 1, o)
                        pltpu.async_copy(z_h.at[gv[o]], rows[o], sem_g[o])

                    pltpu.sync_copy(rows[q], acc.at[dlv[q]], add=True)

                    @pl.when(i + 2 < ntot)
                    def _():
                        rec_load(i + 2, q)

            return 0

        lax.fori_loop(0, lax.div(ntot + 1, jnp.int32(2)), pair, 0)
        plsc.subcore_barrier()
        pltpu.sync_copy(acc.at[pl.ds(s * RPT, RPT)],
                        out_h.at[pl.ds(r * N + lo + s * RPT, RPT)])

        @pl.when(s == 0)
        def _():
            pltpu.sync_copy(acc.at[pl.ds(NS * RPT, REXTRA)],
                            out_h.at[pl.ds(r * N + lo + NS * RPT, REXTRA)])


def _mlp_body(u0, u1, u2, u3, i0, i1, i2, i3, w1_ref, b1_ref, w2_ref, b2_ref,
              out_ref):
    feats = [u0, u1, u2, u3, i0, i1, i2, i3]
    acc = jnp.broadcast_to(b1_ref[...], (1000, 128))
    for k, f in enumerate(feats):
        h = jnp.tanh(f[...])
        acc = acc + _dot16(h, w1_ref[pl.ds(32 * k, 32), :])
    r = jnp.maximum(acc, 0.0)
    o = (jnp.sum(_r16(r) * _r16(w2_ref[...]), axis=1, keepdims=True)
         + b2_ref[0, 0])
    out_ref[...] = o


def _mlp(us, its, w1, b1, w2t, b2):
    nq = N // 4
    specs = [pl.BlockSpec((1000, HID), lambda i: (i, 0))] * 8
    specs += [
        pl.BlockSpec((256, 128), lambda i: (0, 0)),
        pl.BlockSpec((1, 128), lambda i: (0, 0)),
        pl.BlockSpec((1, 128), lambda i: (0, 0)),
        pl.BlockSpec((1, 1), lambda i: (0, 0)),
    ]
    return pl.pallas_call(
        _mlp_body,
        grid=(nq // 1000,),
        in_specs=specs,
        out_specs=pl.BlockSpec((1000, 1), lambda i: (i, 0)),
        out_shape=jax.ShapeDtypeStruct((nq, 1), _f32),
    )(*us, *its, w1, b1, w2t, b2)


def kernel(x, edge_index, edge_type, batch,
           basis0, comp0, root0, bias0, basis1, comp1, root1, bias1,
           basis2, comp2, root2, bias2, basis3, comp3, root3, bias3,
           W1, b1, W2, b2):
    src = edge_index[0]
    dst = edge_index[1]
    npad = E_PAD - E
    src_p = jnp.concatenate([src, jnp.zeros((npad,), _i32)])
    dst_p = jnp.concatenate([dst, jnp.full((npad,), N, _i32)])
    et_p = jnp.concatenate([edge_type, jnp.zeros((npad,), _i32)])

    cnt_part, gidx, tly = _p1_counts(src_p, dst_p, et_p)
    inv = _p2_inv(cnt_part.reshape(2, CNTP // 128, 128)).reshape(CNTP)
    invr = inv[:RN].reshape(R, N, 1)
    segtab = _pb_segtab(tly)
    recg, recd = _pc_partition(gidx, dst_p, et_p, segtab)

    def wprep(basis, comp, root, bias):
        b6 = jnp.concatenate([basis, root[None]], axis=0)
        c6 = jnp.concatenate([
            jnp.concatenate([comp, jnp.zeros((R, 1), _f32)], axis=1),
            jnp.array([[0.0, 0.0, 1.0]], _f32)], axis=0).reshape(6, 1, 3)
        bb = jnp.concatenate([jnp.zeros((R, HID), _f32), bias[None]],
                             axis=0).reshape(6, 1, HID)
        return b6, c6, bb

    layers = [(basis0, comp0, root0, bias0), (basis1, comp1, root1, bias1),
              (basis2, comp2, root2, bias2), (basis3, comp3, root3, bias3)]

    b6, c6, bb = wprep(*layers[0])
    z = _proj_first(x, b6, c6, bb)
    agg = _agg(z, recg, recd, segtab).reshape(R, N, HID)
    pre = []
    for li in (1, 2, 3):
        b6, c6, bb = wprep(*layers[li])
        z_new, p_prev = _proj_fused(z, agg, invr, b6, c6, bb)
        pre.append(p_prev)
        z = z_new
        agg = _agg(z, recg, recd, segtab).reshape(R, N, HID)
    pre.append(_cmb(z, agg, invr))

    nq = N // 4
    us = [p.reshape(nq, 4, HID)[:, 0, :] for p in pre]
    its = [p.reshape(nq, 4, HID)[:, 1, :] for p in pre]
    return _mlp(us, its, W1, b1.reshape(1, 128), W2.reshape(1, 128),
                b2.reshape(1, 1))
